# baseline (speedup 1.0000x reference)
import functools

import jax
import jax.numpy as jnp
from jax import lax
from jax.experimental import pallas as pl
from jax.experimental.pallas import tpu as pltpu
from jax.experimental.pallas import tpu_sc as plsc

NC = 2
NS = 16
NW = NC * NS
L = 16


def _make_mesh():
    return plsc.VectorSubcoreMesh(
        core_axis_name="c", subcore_axis_name="s",
        num_cores=NC, num_subcores=NS)


def _make_deg_kernel(N, E):
    VPT = E // (NW * L)

    @functools.partial(
        pl.kernel, mesh=_make_mesh(),
        out_type=jax.ShapeDtypeStruct((NW * N,), jnp.float32),
        scratch_types=[
            pltpu.VMEM((VPT, L), jnp.int32),
            pltpu.VMEM((N,), jnp.float32),
        ],
        compiler_params=pltpu.CompilerParams(needs_layout_passes=False),
    )
    def deg_kernel(dst_hbm, out_hbm, idx_v, hist_v):
        wid = lax.axis_index("c") * NS + lax.axis_index("s")
        pltpu.sync_copy(dst_hbm.at[wid], idx_v)

        zero = jnp.zeros((L,), jnp.float32)

        def zstep(r, carry):
            hist_v[pl.ds(r * L, L)] = zero
            return carry
        lax.fori_loop(0, N // L, zstep, 0)

        ones = jnp.ones((L,), jnp.float32)

        def step(t, carry):
            plsc.addupdate_scatter(hist_v, [idx_v[t]], ones)
            return carry
        lax.fori_loop(0, VPT, step, 0)

        pltpu.sync_copy(hist_v, out_hbm.at[pl.ds(wid * N, N)])

    return deg_kernel


def _make_agg_kernel(NP, E, B):
    KPT = E // (NW * B)
    RPT = NP // NS

    assert KPT % 2 == 0

    @functools.partial(
        pl.kernel, mesh=_make_mesh(),
        out_type=jax.ShapeDtypeStruct((NC, NP, L), jnp.float32),
        scratch_types=[
            pltpu.VMEM((KPT, B), jnp.int32),
            pltpu.VMEM((KPT, B), jnp.int32),
            pltpu.VMEM((B, L), jnp.float32),
            pltpu.VMEM((B, L), jnp.float32),
            pltpu.VMEM((RPT, L), jnp.float32),
            pltpu.VMEM_SHARED((NP, L), jnp.float32),
            pltpu.VMEM_SHARED((NP, L), jnp.float32),
            pltpu.SemaphoreType.DMA,
            pltpu.SemaphoreType.DMA,
        ],
        compiler_params=pltpu.CompilerParams(
            needs_layout_passes=False, use_tc_tiling_on_sc=False),
    )
    def agg_kernel(hs_hbm, src_hbm, dst_hbm, out_hbm,
                   sidx, didx, rows0, rows1, zbuf, acc, hs_s, sem0, sem1):
        cid = lax.axis_index("c")
        sid = lax.axis_index("s")
        wid = cid * NS + sid

        pltpu.sync_copy(src_hbm.at[wid], sidx)
        pltpu.sync_copy(dst_hbm.at[wid], didx)
        pltpu.sync_copy(hs_hbm.at[pl.ds(sid * RPT, RPT)],
                        hs_s.at[pl.ds(sid * RPT, RPT)])

        zero = jnp.zeros((L,), jnp.float32)

        def zstep(r, carry):
            zbuf[r] = zero
            return carry
        lax.fori_loop(0, RPT, zstep, 0)
        pltpu.sync_copy(zbuf, acc.at[pl.ds(sid * RPT, RPT)])
        plsc.subcore_barrier()

        pltpu.async_copy(hs_s.at[sidx.at[0]], rows0, sem0)
        pltpu.async_copy(hs_s.at[sidx.at[1]], rows1, sem1)

        def step(jj, carry):
            j0 = 2 * jj
            pltpu.make_async_copy(hs_s.at[sidx.at[j0]], rows0,
                                  sem0).wait()
            pltpu.sync_copy(rows0, acc.at[didx.at[j0]], add=True)

            @pl.when(jj < KPT // 2 - 1)
            def _():
                pltpu.async_copy(hs_s.at[sidx.at[j0 + 2]], rows0, sem0)

            pltpu.make_async_copy(hs_s.at[sidx.at[j0 + 1]], rows1,
                                  sem1).wait()
            pltpu.sync_copy(rows1, acc.at[didx.at[j0 + 1]], add=True)

            @pl.when(jj < KPT // 2 - 1)
            def _():
                pltpu.async_copy(hs_s.at[sidx.at[j0 + 3]], rows1, sem1)
            return carry
        lax.fori_loop(0, KPT // 2, step, 0)

        plsc.subcore_barrier()
        pltpu.sync_copy(acc.at[pl.ds(sid * RPT, RPT)],
                        out_hbm.at[cid, pl.ds(sid * RPT, RPT)])

    return agg_kernel


def _dense_mm(x, W1):
    N = x.shape[0]
    H = W1.shape[1]

    def body(x_ref, w_ref, h1_ref):
        h1_ref[...] = jnp.dot(x_ref[...], w_ref[...],
                              preferred_element_type=jnp.float32)

    return pl.pallas_call(
        body,
        out_shape=jax.ShapeDtypeStruct((N, H), jnp.float32),
    )(x, W1)


def _dense1b(deg2d, h1, NP):
    N, H = h1.shape
    NWd = deg2d.shape[0]

    def body(deg_ref, h1_ref, dinv_ref, hs1_ref):
        ones_v = jnp.ones((NWd, 1), jnp.float32)
        deg = lax.dot_general(
            deg_ref[...], ones_v, (((0,), (0,)), ((), ())),
            preferred_element_type=jnp.float32) + 1.0
        dinv = lax.rsqrt(deg)
        dinv_ref[...] = dinv
        hs1_ref[pl.ds(0, N), :] = h1_ref[...] * dinv

    return pl.pallas_call(
        body,
        out_shape=(
            jax.ShapeDtypeStruct((N, 1), jnp.float32),
            jax.ShapeDtypeStruct((NP, H), jnp.float32),
        ),
    )(deg2d, h1)


def _dense2(p1, h1, dinv, b1, NP):
    N, H = h1.shape

    def body(p_ref, h1_ref, dinv_ref, b1_ref, a1_ref, hs2_ref):
        dv = dinv_ref[...]
        psum = p_ref[0, pl.ds(0, N), :] + p_ref[1, pl.ds(0, N), :]
        out1 = dv * psum + dv * dv * h1_ref[...] + b1_ref[...]
        a1 = jnp.maximum(out1, 0.0)
        a1_ref[...] = a1
        hs2_ref[pl.ds(0, N), :] = a1 * dv

    return pl.pallas_call(
        body,
        out_shape=(
            jax.ShapeDtypeStruct((N, H), jnp.float32),
            jax.ShapeDtypeStruct((NP, H), jnp.float32),
        ),
    )(p1, h1, dinv, b1)


def _dense3(p2, a1, dinv, W2, b2):
    N = a1.shape[0]
    C = W2.shape[1]

    def body(p_ref, a1_ref, dinv_ref, w2_ref, b2_ref, out_ref):
        dv = dinv_ref[...]
        psum = p_ref[0, pl.ds(0, N), :] + p_ref[1, pl.ds(0, N), :]
        pre2 = dv * psum + dv * dv * a1_ref[...]
        o = jnp.dot(pre2, w2_ref[...],
                    preferred_element_type=jnp.float32) + b2_ref[...]
        m = jnp.max(o, axis=1, keepdims=True)
        z = o - m
        out_ref[...] = z - jnp.log(
            jnp.sum(jnp.exp(z), axis=1, keepdims=True))

    return pl.pallas_call(
        body,
        out_shape=jax.ShapeDtypeStruct((N, C), jnp.float32),
    )(p2, a1, dinv, W2, b2)


def kernel(x, edge_index, W1, b1, W2, b2):
    N, F = x.shape
    H = W1.shape[1]
    C = W2.shape[1]
    E = edge_index.shape[1]
    B = 125

    assert H == L and N % L == 0 and N % NS == 0 \
        and E % (NW * B) == 0 and E % (NW * L) == 0

    NP = -(-N // (NS * 8)) * (NS * 8)

    src = edge_index[0].astype(jnp.int32)
    dst = edge_index[1].astype(jnp.int32)
    dst16 = dst.reshape(NW, E // (NW * L), L)
    srcB = src.reshape(NW, E // (NW * B), B)
    dstB = dst.reshape(NW, E // (NW * B), B)

    deg_parts = _make_deg_kernel(N, E)(dst16)
    deg2d = deg_parts.reshape(NW, N)

    h1 = _dense_mm(x, W1)
    dinv, hs1 = _dense1b(deg2d, h1, NP)

    agg = _make_agg_kernel(NP, E, B)
    p1 = agg(hs1, srcB, dstB)
    a1, hs2 = _dense2(p1, h1, dinv, b1.reshape(1, H), NP)

    p2 = agg(hs2, srcB, dstB)
    out = _dense3(p2, a1, dinv, W2, b2.reshape(1, C))
    return out


# speedup vs baseline: 1.0264x; 1.0264x over previous
import functools

import jax
import jax.numpy as jnp
from jax import lax
from jax.experimental import pallas as pl
from jax.experimental.pallas import tpu as pltpu
from jax.experimental.pallas import tpu_sc as plsc

NC = 2
NS = 16
NW = NC * NS
L = 16


def _make_mesh():
    return plsc.VectorSubcoreMesh(
        core_axis_name="c", subcore_axis_name="s",
        num_cores=NC, num_subcores=NS)


def _make_deg_kernel(N, E):
    VPT = E // (NW * L)

    @functools.partial(
        pl.kernel, mesh=_make_mesh(),
        out_type=jax.ShapeDtypeStruct((NW * N,), jnp.float32),
        scratch_types=[
            pltpu.VMEM((VPT, L), jnp.int32),
            pltpu.VMEM((N,), jnp.float32),
        ],
        compiler_params=pltpu.CompilerParams(needs_layout_passes=False),
    )
    def deg_kernel(dst_hbm, out_hbm, idx_v, hist_v):
        wid = lax.axis_index("c") * NS + lax.axis_index("s")
        pltpu.sync_copy(dst_hbm.at[wid], idx_v)

        zero = jnp.zeros((L,), jnp.float32)

        def zstep(r, carry):
            hist_v[pl.ds(r * L, L)] = zero
            return carry
        lax.fori_loop(0, N // L, zstep, 0)

        ones = jnp.ones((L,), jnp.float32)

        def step(t, carry):
            plsc.addupdate_scatter(hist_v, [idx_v[t]], ones)
            return carry
        lax.fori_loop(0, VPT, step, 0)

        pltpu.sync_copy(hist_v, out_hbm.at[pl.ds(wid * N, N)])

    return deg_kernel


def _make_agg_kernel(NP, E, B):
    KPT = E // (NW * B)
    RPT = NP // NS

    assert KPT % 2 == 0

    @functools.partial(
        pl.kernel, mesh=_make_mesh(),
        out_type=jax.ShapeDtypeStruct((NC, NP, L), jnp.float32),
        scratch_types=[
            pltpu.VMEM((KPT, B), jnp.int32),
            pltpu.VMEM((KPT, B), jnp.int32),
            pltpu.VMEM((B, L), jnp.float32),
            pltpu.VMEM((B, L), jnp.float32),
            pltpu.VMEM((RPT, L), jnp.float32),
            pltpu.VMEM_SHARED((NP, L), jnp.float32),
            pltpu.VMEM_SHARED((NP, L), jnp.float32),
            pltpu.SemaphoreType.DMA,
            pltpu.SemaphoreType.DMA,
        ],
        compiler_params=pltpu.CompilerParams(
            needs_layout_passes=False, use_tc_tiling_on_sc=False),
    )
    def agg_kernel(hs_hbm, src_hbm, dst_hbm, out_hbm,
                   sidx, didx, rows0, rows1, zbuf, acc, hs_s, sem0, sem1):
        cid = lax.axis_index("c")
        sid = lax.axis_index("s")
        wid = cid * NS + sid

        pltpu.sync_copy(src_hbm.at[wid], sidx)
        pltpu.sync_copy(dst_hbm.at[wid], didx)
        pltpu.sync_copy(hs_hbm.at[pl.ds(sid * RPT, RPT)],
                        hs_s.at[pl.ds(sid * RPT, RPT)])

        zero = jnp.zeros((L,), jnp.float32)

        def zstep(r, carry):
            zbuf[r] = zero
            return carry
        lax.fori_loop(0, RPT, zstep, 0)
        pltpu.sync_copy(zbuf, acc.at[pl.ds(sid * RPT, RPT)])
        plsc.subcore_barrier()

        pltpu.async_copy(hs_s.at[sidx.at[0]], rows0, sem0)
        pltpu.async_copy(hs_s.at[sidx.at[1]], rows1, sem1)

        def step(jj, carry):
            j0 = 2 * jj
            pltpu.make_async_copy(hs_s.at[sidx.at[j0]], rows0,
                                  sem0).wait()
            pltpu.sync_copy(rows0, acc.at[didx.at[j0]], add=True)

            @pl.when(jj < KPT // 2 - 1)
            def _():
                pltpu.async_copy(hs_s.at[sidx.at[j0 + 2]], rows0, sem0)

            pltpu.make_async_copy(hs_s.at[sidx.at[j0 + 1]], rows1,
                                  sem1).wait()
            pltpu.sync_copy(rows1, acc.at[didx.at[j0 + 1]], add=True)

            @pl.when(jj < KPT // 2 - 1)
            def _():
                pltpu.async_copy(hs_s.at[sidx.at[j0 + 3]], rows1, sem1)
            return carry
        lax.fori_loop(0, KPT // 2, step, 0)

        plsc.subcore_barrier()
        pltpu.sync_copy(acc.at[pl.ds(sid * RPT, RPT)],
                        out_hbm.at[cid, pl.ds(sid * RPT, RPT)])

    return agg_kernel


def _dense_mm(x, W1):
    N = x.shape[0]
    H = W1.shape[1]

    def body(x_ref, w_ref, h1_ref):
        h1_ref[...] = jnp.dot(x_ref[...], w_ref[...],
                              preferred_element_type=jnp.float32)

    return pl.pallas_call(
        body,
        out_shape=jax.ShapeDtypeStruct((N, H), jnp.float32),
    )(x, W1)


def _dense1b(deg2d, h1, NP):
    N, H = h1.shape
    NWd = deg2d.shape[0]

    def body(deg_ref, h1_ref, dinv_ref, hs1_ref):
        ones_v = jnp.ones((NWd, 1), jnp.float32)
        deg = lax.dot_general(
            deg_ref[...], ones_v, (((0,), (0,)), ((), ())),
            preferred_element_type=jnp.float32) + 1.0
        dinv = lax.rsqrt(deg)
        dinv_ref[...] = dinv
        hs1_ref[pl.ds(0, N), :] = h1_ref[...] * dinv

    return pl.pallas_call(
        body,
        out_shape=(
            jax.ShapeDtypeStruct((N, 1), jnp.float32),
            jax.ShapeDtypeStruct((NP, H), jnp.float32),
        ),
    )(deg2d, h1)


def _dense2(p1, hs1, dinv, b1, N):
    NP, H = hs1.shape

    def body(p_ref, hs1_ref, dinv_ref, b1_ref, hs2_ref):
        dv = dinv_ref[...]
        psum = p_ref[0, pl.ds(0, N), :] + p_ref[1, pl.ds(0, N), :] \
            + hs1_ref[pl.ds(0, N), :]
        a1 = jnp.maximum(dv * psum + b1_ref[...], 0.0)
        hs2_ref[pl.ds(0, N), :] = a1 * dv

    return pl.pallas_call(
        body,
        out_shape=jax.ShapeDtypeStruct((NP, H), jnp.float32),
    )(p1, hs1, dinv, b1)


def _dense3(p2, hs2, dinv, W2, b2, N):
    C = W2.shape[1]

    def body(p_ref, hs2_ref, dinv_ref, w2_ref, b2_ref, out_ref):
        dv = dinv_ref[...]
        psum = p_ref[0, pl.ds(0, N), :] + p_ref[1, pl.ds(0, N), :] \
            + hs2_ref[pl.ds(0, N), :]
        pre2 = dv * psum
        o = jnp.dot(pre2, w2_ref[...],
                    preferred_element_type=jnp.float32) + b2_ref[...]
        m = jnp.max(o, axis=1, keepdims=True)
        z = o - m
        out_ref[...] = z - jnp.log(
            jnp.sum(jnp.exp(z), axis=1, keepdims=True))

    return pl.pallas_call(
        body,
        out_shape=jax.ShapeDtypeStruct((N, C), jnp.float32),
    )(p2, hs2, dinv, W2, b2)


def kernel(x, edge_index, W1, b1, W2, b2):
    N, F = x.shape
    H = W1.shape[1]
    C = W2.shape[1]
    E = edge_index.shape[1]
    B = 125

    assert H == L and N % L == 0 and N % NS == 0 \
        and E % (NW * B) == 0 and E % (NW * L) == 0

    NP = -(-N // (NS * 8)) * (NS * 8)

    src = edge_index[0].astype(jnp.int32)
    dst = edge_index[1].astype(jnp.int32)
    dst16 = dst.reshape(NW, E // (NW * L), L)
    srcB = src.reshape(NW, E // (NW * B), B)
    dstB = dst.reshape(NW, E // (NW * B), B)

    deg_parts = _make_deg_kernel(N, E)(dst16)
    deg2d = deg_parts.reshape(NW, N)

    h1 = _dense_mm(x, W1)
    dinv, hs1 = _dense1b(deg2d, h1, NP)

    agg = _make_agg_kernel(NP, E, B)
    p1 = agg(hs1, srcB, dstB)
    hs2 = _dense2(p1, hs1, dinv, b1.reshape(1, H), N)

    p2 = agg(hs2, srcB, dstB)
    out = _dense3(p2, hs2, dinv, W2, b2.reshape(1, C), N)
    return out


# speedup vs baseline: 1.0349x; 1.0082x over previous
import functools

import jax
import jax.numpy as jnp
from jax import lax
from jax.experimental import pallas as pl
from jax.experimental.pallas import tpu as pltpu
from jax.experimental.pallas import tpu_sc as plsc

NC = 2
NS = 16
NW = NC * NS
L = 16


def _make_mesh():
    return plsc.VectorSubcoreMesh(
        core_axis_name="c", subcore_axis_name="s",
        num_cores=NC, num_subcores=NS)


def _make_deg_kernel(N, E):
    VPT = E // (NW * L)

    @functools.partial(
        pl.kernel, mesh=_make_mesh(),
        out_type=jax.ShapeDtypeStruct((NW * N,), jnp.float32),
        scratch_types=[
            pltpu.VMEM((VPT, L), jnp.int32),
            pltpu.VMEM((N,), jnp.float32),
        ],
        compiler_params=pltpu.CompilerParams(needs_layout_passes=False),
    )
    def deg_kernel(dst_hbm, out_hbm, idx_v, hist_v):
        wid = lax.axis_index("c") * NS + lax.axis_index("s")
        pltpu.sync_copy(dst_hbm.at[wid], idx_v)

        zero = jnp.zeros((L,), jnp.float32)

        def zstep(r, carry):
            hist_v[pl.ds(r * L, L)] = zero
            return carry
        lax.fori_loop(0, N // L, zstep, 0)

        ones = jnp.ones((L,), jnp.float32)

        def step(t, carry):
            plsc.addupdate_scatter(hist_v, [idx_v[t]], ones)
            return carry
        lax.fori_loop(0, VPT, step, 0)

        pltpu.sync_copy(hist_v, out_hbm.at[pl.ds(wid * N, N)])

    return deg_kernel


def _make_agg_kernel(NP, E, B):
    KPT = E // (NW * B)
    RPT = NP // NS

    assert KPT % 2 == 0

    @functools.partial(
        pl.kernel, mesh=_make_mesh(),
        out_type=jax.ShapeDtypeStruct((NC, NP, L), jnp.float32),
        scratch_types=[
            pltpu.VMEM((KPT, B), jnp.int32),
            pltpu.VMEM((KPT, B), jnp.int32),
            pltpu.VMEM((B, L), jnp.float32),
            pltpu.VMEM((B, L), jnp.float32),
            pltpu.VMEM((RPT, L), jnp.float32),
            pltpu.VMEM_SHARED((NP, L), jnp.float32),
            pltpu.VMEM_SHARED((NP, L), jnp.float32),
            pltpu.SemaphoreType.DMA,
            pltpu.SemaphoreType.DMA,
        ],
        compiler_params=pltpu.CompilerParams(
            needs_layout_passes=False, use_tc_tiling_on_sc=False),
    )
    def agg_kernel(hs_hbm, src_hbm, dst_hbm, out_hbm,
                   sidx, didx, rows0, rows1, zbuf, acc, hs_s, sem0, sem1):
        cid = lax.axis_index("c")
        sid = lax.axis_index("s")
        wid = cid * NS + sid

        pltpu.sync_copy(src_hbm.at[wid], sidx)
        pltpu.sync_copy(dst_hbm.at[wid], didx)
        pltpu.sync_copy(hs_hbm.at[pl.ds(sid * RPT, RPT)],
                        hs_s.at[pl.ds(sid * RPT, RPT)])

        zero = jnp.zeros((L,), jnp.float32)

        def zstep(r, carry):
            zbuf[r] = zero
            return carry
        lax.fori_loop(0, RPT, zstep, 0)
        pltpu.sync_copy(zbuf, acc.at[pl.ds(sid * RPT, RPT)])
        plsc.subcore_barrier()

        pltpu.async_copy(hs_s.at[sidx.at[0]], rows0, sem0)
        pltpu.async_copy(hs_s.at[sidx.at[1]], rows1, sem1)

        def step(jj, carry):
            j0 = 2 * jj
            pltpu.make_async_copy(hs_s.at[sidx.at[j0]], rows0,
                                  sem0).wait()
            pltpu.sync_copy(rows0, acc.at[didx.at[j0]], add=True)

            @pl.when(jj < KPT // 2 - 1)
            def _():
                pltpu.async_copy(hs_s.at[sidx.at[j0 + 2]], rows0, sem0)

            pltpu.make_async_copy(hs_s.at[sidx.at[j0 + 1]], rows1,
                                  sem1).wait()
            pltpu.sync_copy(rows1, acc.at[didx.at[j0 + 1]], add=True)

            @pl.when(jj < KPT // 2 - 1)
            def _():
                pltpu.async_copy(hs_s.at[sidx.at[j0 + 3]], rows1, sem1)
            return carry
        lax.fori_loop(0, KPT // 2, step, 0)

        plsc.subcore_barrier()
        pltpu.sync_copy(acc.at[pl.ds(sid * RPT, RPT)],
                        out_hbm.at[cid, pl.ds(sid * RPT, RPT)])

    return agg_kernel


def _dense1(deg2d, x, W1, NP):
    N = x.shape[0]
    H = W1.shape[1]
    NWd = deg2d.shape[0]

    def body(deg_ref, x_ref, w_ref, dinv_ref, hs1_ref):
        ones_v = jnp.ones((NWd, 1), jnp.float32)
        deg = lax.dot_general(
            deg_ref[...], ones_v, (((0,), (0,)), ((), ())),
            preferred_element_type=jnp.float32) + 1.0
        dinv = lax.rsqrt(deg)
        h1 = jnp.dot(x_ref[...], w_ref[...],
                     preferred_element_type=jnp.float32)
        dinv_ref[...] = dinv
        hs1_ref[pl.ds(0, N), :] = h1 * dinv

    return pl.pallas_call(
        body,
        out_shape=(
            jax.ShapeDtypeStruct((N, 1), jnp.float32),
            jax.ShapeDtypeStruct((NP, H), jnp.float32),
        ),
    )(deg2d, x, W1)


def _dense2(p1, hs1, dinv, b1, N):
    NP, H = hs1.shape

    def body(p_ref, hs1_ref, dinv_ref, b1_ref, hs2_ref):
        dv = dinv_ref[...]
        psum = p_ref[0, pl.ds(0, N), :] + p_ref[1, pl.ds(0, N), :] \
            + hs1_ref[pl.ds(0, N), :]
        a1 = jnp.maximum(dv * psum + b1_ref[...], 0.0)
        hs2_ref[pl.ds(0, N), :] = a1 * dv

    return pl.pallas_call(
        body,
        out_shape=jax.ShapeDtypeStruct((NP, H), jnp.float32),
    )(p1, hs1, dinv, b1)


def _dense3(p2, hs2, dinv, W2, b2, N):
    C = W2.shape[1]

    def body(p_ref, hs2_ref, dinv_ref, w2_ref, b2_ref, out_ref):
        dv = dinv_ref[...]
        psum = p_ref[0, pl.ds(0, N), :] + p_ref[1, pl.ds(0, N), :] \
            + hs2_ref[pl.ds(0, N), :]
        pre2 = dv * psum
        o = jnp.dot(pre2, w2_ref[...],
                    preferred_element_type=jnp.float32) + b2_ref[...]
        m = jnp.max(o, axis=1, keepdims=True)
        z = o - m
        out_ref[...] = z - jnp.log(
            jnp.sum(jnp.exp(z), axis=1, keepdims=True))

    return pl.pallas_call(
        body,
        out_shape=jax.ShapeDtypeStruct((N, C), jnp.float32),
    )(p2, hs2, dinv, W2, b2)


def kernel(x, edge_index, W1, b1, W2, b2):
    N, F = x.shape
    H = W1.shape[1]
    C = W2.shape[1]
    E = edge_index.shape[1]
    B = 125

    assert H == L and N % L == 0 and N % NS == 0 \
        and E % (NW * B) == 0 and E % (NW * L) == 0

    NP = -(-N // (NS * 8)) * (NS * 8)

    src = edge_index[0].astype(jnp.int32)
    dst = edge_index[1].astype(jnp.int32)
    dst16 = dst.reshape(NW, E // (NW * L), L)
    srcB = src.reshape(NW, E // (NW * B), B)
    dstB = dst.reshape(NW, E // (NW * B), B)

    deg_parts = _make_deg_kernel(N, E)(dst16)
    deg2d = deg_parts.reshape(NW, N)

    dinv, hs1 = _dense1(deg2d, x, W1, NP)

    agg = _make_agg_kernel(NP, E, B)
    p1 = agg(hs1, srcB, dstB)
    hs2 = _dense2(p1, hs1, dinv, b1.reshape(1, H), N)

    p2 = agg(hs2, srcB, dstB)
    out = _dense3(p2, hs2, dinv, W2, b2.reshape(1, C), N)
    return out
